# baseline (speedup 1.0000x reference)
import functools

import jax
import jax.numpy as jnp
from jax import lax
from jax.experimental import pallas as pl
from jax.experimental.pallas import tpu as pltpu
from jax.experimental.pallas import tpu_sc as plsc

N = 10000
E = 320000
D = 128
HD = D // 2
NC, NS = 2, 16
NW = NC * NS
CH = 80
NB = 10
ECS = E // NS
NCHUNK = ECS // CH
EPD = E // NW
NCHUNK_D = EPD // CH
NPAD = 10240
RPT = NPAD // NS

_mesh = plsc.VectorSubcoreMesh(core_axis_name="c", subcore_axis_name="s")


def _deg_body(dst3_hbm, w3_hbm, out_hbm, dstall, wall, zv, dsem, degsp):
    c = lax.axis_index("c")
    s = lax.axis_index("s")
    wid = s * NC + c
    for i in range(RPT // 16):
        zv[pl.ds(i * 16, 16)] = jnp.zeros((16,), jnp.float32)
    pltpu.sync_copy(zv, degsp.at[pl.ds(s * RPT, RPT)])
    pltpu.sync_copy(dst3_hbm.at[wid], dstall)
    pltpu.sync_copy(w3_hbm.at[wid], wall)
    plsc.subcore_barrier()

    def fire(g, _):
        pltpu.async_copy(wall.at[g], degsp.at[dstall.at[g]], dsem, add=True)

        @pl.when(g >= 16)
        def _wait():
            pltpu.make_async_copy(wall.at[0], degsp.at[pl.ds(0, CH)],
                                  dsem).wait()
        return 0

    lax.fori_loop(0, NCHUNK_D, fire, 0)
    for _ in range(16):
        pltpu.make_async_copy(wall.at[0], degsp.at[pl.ds(0, CH)], dsem).wait()
    plsc.subcore_barrier()
    pltpu.sync_copy(degsp.at[pl.ds(s * RPT, RPT)],
                    out_hbm.at[pl.ds(c * NPAD + s * RPT, RPT)])


_deg_call = functools.partial(
    pl.kernel,
    out_type=jax.ShapeDtypeStruct((NC * NPAD,), jnp.float32),
    mesh=_mesh,
    scratch_types=[
        pltpu.VMEM((NCHUNK_D, CH), jnp.int32),
        pltpu.VMEM((NCHUNK_D, CH), jnp.float32),
        pltpu.VMEM((RPT,), jnp.float32),
        pltpu.SemaphoreType.DMA,
        pltpu.VMEM_SHARED((NPAD,), jnp.float32),
    ],
)(_deg_body)


def _prop_body(xs2_hbm, src_hbm, dst_hbm, w_hbm, out_hbm, *rest):
    sv = rest[0:NB]
    dv = rest[NB:2 * NB]
    wv = rest[2 * NB:3 * NB]
    rows = rest[3 * NB:4 * NB]
    isem = rest[4 * NB:5 * NB]
    gsem = rest[5 * NB:6 * NB]
    ssem = rest[6 * NB:7 * NB]
    accsp = rest[7 * NB]
    c = lax.axis_index("c")
    s = lax.axis_index("s")
    r0 = s * RPT
    e0 = s * ECS
    pltpu.sync_copy(xs2_hbm.at[c, pl.ds(r0, RPT)], accsp.at[pl.ds(r0, RPT)])

    def start_idx(g, b):
        base = e0 + g * CH
        pltpu.async_copy(src_hbm.at[pl.ds(base, CH)], sv[b], isem[b])
        pltpu.async_copy(dst_hbm.at[pl.ds(base, CH)], dv[b], isem[b])
        pltpu.async_copy(w_hbm.at[pl.ds(base, CH)], wv[b], isem[b])

    def wait_idx(b):
        for _ in range(3):
            pltpu.make_async_copy(src_hbm.at[pl.ds(0, CH)], sv[b],
                                  isem[b]).wait()

    def wait_rows_sem(sem, b):
        pltpu.make_async_copy(xs2_hbm.at[c, pl.ds(0, CH)], rows[b],
                              sem[b]).wait()

    start_idx(0, 0)
    start_idx(1, 1)
    plsc.subcore_barrier()
    wait_idx(0)
    pltpu.async_copy(xs2_hbm.at[c].at[sv[0]], rows[0], gsem[0])

    def scale(b, g):
        del g

        def sixteen(t, _):
            wrow = wv[b][pl.ds(t * 16, 16)]
            for l in range(16):
                wj = wrow[l]
                j = t * 16 + l
                for k in range(HD // 16):
                    sl = pl.ds(k * 16, 16)
                    rows[b][j, sl] = rows[b][j, sl] * wj
            return 0

        lax.fori_loop(0, CH // 16, sixteen, 0)

    def group(grp, _):
        for b in range(NB):
            g = grp * NB + b
            b2 = (b + 1) % NB
            b3 = (b + 2) % NB

            @pl.when(g + 2 < NCHUNK)
            def _prefetch_idx():
                @pl.when(g >= NB - 2)
                def _free():
                    wait_rows_sem(ssem, b3)
                start_idx(g + 2, b3)

            @pl.when(g + 1 < NCHUNK)
            def _prefetch_rows():
                wait_idx(b2)
                pltpu.async_copy(xs2_hbm.at[c].at[sv[b2]], rows[b2],
                                 gsem[b2])

            wait_rows_sem(gsem, b)
            scale(b, g)
            pltpu.async_copy(rows[b], accsp.at[dv[b]], ssem[b], add=True)
        return 0

    lax.fori_loop(0, NCHUNK // NB, group, 0)
    for b in range(NB):
        wait_rows_sem(ssem, b)
    plsc.subcore_barrier()
    pltpu.sync_copy(accsp.at[pl.ds(r0, RPT)],
                    out_hbm.at[c, pl.ds(r0, RPT)])


_prop_call = functools.partial(
    pl.kernel,
    out_type=jax.ShapeDtypeStruct((NC, NPAD, HD), jnp.float32),
    mesh=_mesh,
    scratch_types=(
        [pltpu.VMEM((CH,), jnp.int32) for _ in range(NB)]
        + [pltpu.VMEM((CH,), jnp.int32) for _ in range(NB)]
        + [pltpu.VMEM((CH,), jnp.float32) for _ in range(NB)]
        + [pltpu.VMEM((CH, HD), jnp.float32) for _ in range(NB)]
        + [pltpu.SemaphoreType.DMA for _ in range(3 * NB)]
        + [pltpu.VMEM_SHARED((NPAD, HD), jnp.float32)]
    ),
    compiler_params=pltpu.CompilerParams(use_tc_tiling_on_sc=False),
)(_prop_body)


def _tc1_body(y_ref, w1_ref, degp_ref, xs2_ref, dinv_ref):
    xw = jnp.dot(y_ref[...], w1_ref[...], preferred_element_type=jnp.float32)
    deg = degp_ref[0, :N, :] + degp_ref[1, :N, :] + 1.0
    dinv = jnp.where(deg > 0, lax.rsqrt(deg), 0.0)
    xs = xw * dinv
    zpad = jnp.zeros((NPAD - N, HD), jnp.float32)
    xs2_ref[0, :N, :] = xs[:, :HD]
    xs2_ref[0, pl.ds(N, NPAD - N), :] = zpad
    xs2_ref[1, :N, :] = xs[:, HD:]
    xs2_ref[1, pl.ds(N, NPAD - N), :] = zpad
    dinv_ref[...] = dinv


def _tc1(Y, W1, degp3):
    return pl.pallas_call(
        _tc1_body,
        out_shape=[
            jax.ShapeDtypeStruct((NC, NPAD, HD), jnp.float32),
            jax.ShapeDtypeStruct((N, 1), jnp.float32),
        ],
    )(Y, W1, degp3)


def _tc2_body(acc_ref, dinv_ref, b1_ref, hs2_ref):
    dinv = dinv_ref[...]
    zpad = jnp.zeros((NPAD - N, HD), jnp.float32)
    for h in range(NC):
        s1 = acc_ref[h, :N, :]
        hh = jax.nn.relu(s1 * dinv + b1_ref[pl.ds(h * HD, HD)])
        hs2_ref[h, :N, :] = hh * dinv
        hs2_ref[h, pl.ds(N, NPAD - N), :] = zpad


def _tc2(acc, dinv, b1):
    return pl.pallas_call(
        _tc2_body,
        out_shape=jax.ShapeDtypeStruct((NC, NPAD, HD), jnp.float32),
    )(acc, dinv, b1)


def _tc3_body(acc_ref, dinv_ref, wmu_ref, bmu_ref, wlv_ref, blv_ref,
              mu_ref, lv_ref):
    dinv = dinv_ref[...]
    p_lo = acc_ref[0, :N, :] * dinv
    p_hi = acc_ref[1, :N, :] * dinv
    mu_ref[...] = (
        jnp.dot(p_lo, wmu_ref[:HD, :], preferred_element_type=jnp.float32)
        + jnp.dot(p_hi, wmu_ref[pl.ds(HD, HD), :],
                  preferred_element_type=jnp.float32)
        + bmu_ref[...])
    lv_ref[...] = (
        jnp.dot(p_lo, wlv_ref[:HD, :], preferred_element_type=jnp.float32)
        + jnp.dot(p_hi, wlv_ref[pl.ds(HD, HD), :],
                  preferred_element_type=jnp.float32)
        + blv_ref[...])


def _tc3(acc, dinv, W_mu, b_mu, W_lv, b_lv):
    lat = W_mu.shape[1]
    return pl.pallas_call(
        _tc3_body,
        out_shape=[
            jax.ShapeDtypeStruct((N, lat), jnp.float32),
            jax.ShapeDtypeStruct((N, lat), jnp.float32),
        ],
    )(acc, dinv, W_mu, b_mu, W_lv, b_lv)


@jax.jit
def kernel(Y, edge_index, edge_weight, W1, b1, W_mu, b_mu, W_lv, b_lv):
    src = edge_index[0].astype(jnp.int32)
    dst = edge_index[1].astype(jnp.int32)
    w = edge_weight.astype(jnp.float32)
    dst3 = dst.reshape(NW, NCHUNK_D, CH)
    w3 = w.reshape(NW, NCHUNK_D, CH)

    degp = _deg_call(dst3, w3)
    degp3 = degp.reshape(NC, NPAD, 1)
    xs2, dinv = _tc1(Y, W1, degp3)
    acc1 = _prop_call(xs2, src, dst, w)
    hs2 = _tc2(acc1, dinv, b1)
    acc2 = _prop_call(hs2, src, dst, w)
    mu, lv = _tc3(acc2, dinv, W_mu, b_mu, W_lv, b_lv)
    return (mu, lv)


# speedup vs baseline: 1.0061x; 1.0061x over previous
import functools

import jax
import jax.numpy as jnp
from jax import lax
from jax.experimental import pallas as pl
from jax.experimental.pallas import tpu as pltpu
from jax.experimental.pallas import tpu_sc as plsc

N = 10000
E = 320000
D = 128
HD = D // 2
NC, NS = 2, 16
NW = NC * NS
CH = 80
NB = 5
ECS = E // NS
NCHUNK = ECS // CH
EPD = E // NW
NCHUNK_D = EPD // CH
NPAD = 10240
RPT = NPAD // NS
HBLK = 128

_mesh = plsc.VectorSubcoreMesh(core_axis_name="c", subcore_axis_name="s")


def _deg_body(dst3_hbm, w3_hbm, out_hbm, dstall, wall, zv, dsem, degsp):
    c = lax.axis_index("c")
    s = lax.axis_index("s")
    wid = s * NC + c
    for i in range(RPT // 16):
        zv[pl.ds(i * 16, 16)] = jnp.zeros((16,), jnp.float32)
    pltpu.sync_copy(zv, degsp.at[pl.ds(s * RPT, RPT)])
    pltpu.sync_copy(dst3_hbm.at[wid], dstall)
    pltpu.sync_copy(w3_hbm.at[wid], wall)
    plsc.subcore_barrier()

    def fire(g, _):
        pltpu.async_copy(wall.at[g], degsp.at[dstall.at[g]], dsem, add=True)

        @pl.when(g >= 16)
        def _wait():
            pltpu.make_async_copy(wall.at[0], degsp.at[pl.ds(0, CH)],
                                  dsem).wait()
        return 0

    lax.fori_loop(0, NCHUNK_D, fire, 0)
    for _ in range(16):
        pltpu.make_async_copy(wall.at[0], degsp.at[pl.ds(0, CH)], dsem).wait()
    plsc.subcore_barrier()
    pltpu.sync_copy(degsp.at[pl.ds(s * RPT, RPT)],
                    out_hbm.at[pl.ds(c * NPAD + s * RPT, RPT)])


_deg_call = functools.partial(
    pl.kernel,
    out_type=jax.ShapeDtypeStruct((NC * NPAD,), jnp.float32),
    mesh=_mesh,
    scratch_types=[
        pltpu.VMEM((NCHUNK_D, CH), jnp.int32),
        pltpu.VMEM((NCHUNK_D, CH), jnp.float32),
        pltpu.VMEM((RPT,), jnp.float32),
        pltpu.SemaphoreType.DMA,
        pltpu.VMEM_SHARED((NPAD,), jnp.float32),
    ],
)(_deg_body)


def _fused_body(xs2_hbm, dinv_hbm, b1_hbm, pk_hbm, wp_hbm, out_hbm, hs_hbm,
                *rest):
    pkv = rest[0:NB]
    wv = rest[NB:2 * NB]
    rows = rest[2 * NB:3 * NB]
    isem = rest[3 * NB:4 * NB]
    gsem = rest[4 * NB:5 * NB]
    ssem = rest[5 * NB:6 * NB]
    dl, b1v, hb, accsp = rest[6 * NB:6 * NB + 4]
    c = lax.axis_index("c")
    s = lax.axis_index("s")
    r0 = s * RPT

    def start_idx(g, b):
        pltpu.async_copy(pk_hbm.at[s, g], pkv[b], isem[b])
        pltpu.async_copy(wp_hbm.at[s, g], wv[b], isem[b])

    def wait_idx(b):
        pltpu.make_async_copy(pk_hbm.at[s, 0], pkv[b], isem[b]).wait()
        pltpu.make_async_copy(wp_hbm.at[s, 0], wv[b], isem[b]).wait()

    def wait_rows_sem(sem, b):
        pltpu.make_async_copy(xs2_hbm.at[0, pl.ds(0, CH)], rows[b],
                              sem[b]).wait()

    def scale(b):
        def sixteen(t, _):
            wrow = wv[b][pl.ds(t * 16, 16)]
            for l in range(16):
                wj = wrow[l]
                j = t * 16 + l
                for k in range(HD // 16):
                    sl = pl.ds(k * 16, 16)
                    rows[b][j, sl] = rows[b][j, sl] * wj
            return 0

        lax.fori_loop(0, CH // 16, sixteen, 0)

    def run_pass(table_hbm):
        start_idx(0, 0)
        start_idx(1, 1)
        wait_idx(0)
        pltpu.async_copy(table_hbm.at[pkv[0].at[0]], rows[0], gsem[0])

        def group(grp, _):
            for b in range(NB):
                g = grp * NB + b
                b2 = (b + 1) % NB
                b3 = (b + 2) % NB

                @pl.when(g + 2 < NCHUNK)
                def _prefetch_idx():
                    @pl.when(g >= NB - 2)
                    def _free():
                        wait_rows_sem(ssem, b3)
                    start_idx(g + 2, b3)

                @pl.when(g + 1 < NCHUNK)
                def _prefetch_rows():
                    wait_idx(b2)
                    pltpu.async_copy(table_hbm.at[pkv[b2].at[0]], rows[b2],
                                     gsem[b2])

                wait_rows_sem(gsem, b)
                scale(b)
                pltpu.async_copy(rows[b], accsp.at[pkv[b].at[1]], ssem[b],
                                 add=True)
            return 0

        lax.fori_loop(0, NCHUNK // NB, group, 0)
        for b in range(NB):
            wait_rows_sem(ssem, b)

    pltpu.sync_copy(dinv_hbm.at[pl.ds(r0, RPT)], dl)
    pltpu.sync_copy(b1_hbm.at[pl.ds(c * HD, HD)], b1v)
    pltpu.sync_copy(xs2_hbm.at[c, pl.ds(r0, RPT)], accsp.at[pl.ds(r0, RPT)])
    plsc.subcore_barrier()

    run_pass(xs2_hbm.at[c])
    plsc.subcore_barrier()

    for blk in range(RPT // HBLK):
        rowbase = r0 + blk * HBLK
        pltpu.sync_copy(accsp.at[pl.ds(rowbase, HBLK)], hb)

        def rowloop(t, _):
            dref = dl[pl.ds(blk * HBLK + t * 16, 16)]
            for l in range(16):
                dj = dref[l]
                j = t * 16 + l
                for k in range(HD // 16):
                    sl = pl.ds(k * 16, 16)
                    v = hb[j, sl] * dj + b1v[sl]
                    hb[j, sl] = jnp.maximum(v, 0.0) * dj
            return 0

        lax.fori_loop(0, HBLK // 16, rowloop, 0)
        pltpu.sync_copy(hb, hs_hbm.at[c, pl.ds(rowbase, HBLK)])
        pltpu.sync_copy(hb, accsp.at[pl.ds(rowbase, HBLK)])
    plsc.subcore_barrier()

    run_pass(hs_hbm.at[c])
    plsc.subcore_barrier()
    pltpu.sync_copy(accsp.at[pl.ds(r0, RPT)],
                    out_hbm.at[c, pl.ds(r0, RPT)])


_fused_call = functools.partial(
    pl.kernel,
    out_type=(
        jax.ShapeDtypeStruct((NC, NPAD, HD), jnp.float32),
        jax.ShapeDtypeStruct((NC, NPAD, HD), jnp.float32),
    ),
    mesh=_mesh,
    scratch_types=(
        [pltpu.VMEM((2, CH), jnp.int32) for _ in range(NB)]
        + [pltpu.VMEM((CH,), jnp.float32) for _ in range(NB)]
        + [pltpu.VMEM((CH, HD), jnp.float32) for _ in range(NB)]
        + [pltpu.SemaphoreType.DMA for _ in range(3 * NB)]
        + [
            pltpu.VMEM((RPT,), jnp.float32),
            pltpu.VMEM((HD,), jnp.float32),
            pltpu.VMEM((HBLK, HD), jnp.float32),
            pltpu.VMEM_SHARED((NPAD, HD), jnp.float32),
        ]
    ),
    compiler_params=pltpu.CompilerParams(use_tc_tiling_on_sc=False),
)(_fused_body)


def _tc1_body(y_ref, w1_ref, degp_ref, xs2_ref, dinv_ref):
    xw = jnp.dot(y_ref[...], w1_ref[...], preferred_element_type=jnp.float32)
    deg = degp_ref[0] + degp_ref[1] + 1.0
    dinv = jnp.where(deg > 0, lax.rsqrt(deg), 0.0)
    xs = xw * dinv[:N, :]
    zpad = jnp.zeros((NPAD - N, HD), jnp.float32)
    xs2_ref[0, :N, :] = xs[:, :HD]
    xs2_ref[0, pl.ds(N, NPAD - N), :] = zpad
    xs2_ref[1, :N, :] = xs[:, HD:]
    xs2_ref[1, pl.ds(N, NPAD - N), :] = zpad
    dinv_ref[...] = dinv


def _tc1(Y, W1, degp3):
    return pl.pallas_call(
        _tc1_body,
        out_shape=[
            jax.ShapeDtypeStruct((NC, NPAD, HD), jnp.float32),
            jax.ShapeDtypeStruct((NPAD, 1), jnp.float32),
        ],
    )(Y, W1, degp3)


def _tc3_body(acc_ref, dinv_ref, wmu_ref, bmu_ref, wlv_ref, blv_ref,
              mu_ref, lv_ref):
    dinv = dinv_ref[:N, :]
    p_lo = acc_ref[0, :N, :] * dinv
    p_hi = acc_ref[1, :N, :] * dinv
    mu_ref[...] = (
        jnp.dot(p_lo, wmu_ref[:HD, :], preferred_element_type=jnp.float32)
        + jnp.dot(p_hi, wmu_ref[pl.ds(HD, HD), :],
                  preferred_element_type=jnp.float32)
        + bmu_ref[...])
    lv_ref[...] = (
        jnp.dot(p_lo, wlv_ref[:HD, :], preferred_element_type=jnp.float32)
        + jnp.dot(p_hi, wlv_ref[pl.ds(HD, HD), :],
                  preferred_element_type=jnp.float32)
        + blv_ref[...])


def _tc3(acc, dinv, W_mu, b_mu, W_lv, b_lv):
    lat = W_mu.shape[1]
    return pl.pallas_call(
        _tc3_body,
        out_shape=[
            jax.ShapeDtypeStruct((N, lat), jnp.float32),
            jax.ShapeDtypeStruct((N, lat), jnp.float32),
        ],
    )(acc, dinv, W_mu, b_mu, W_lv, b_lv)


@jax.jit
def kernel(Y, edge_index, edge_weight, W1, b1, W_mu, b_mu, W_lv, b_lv):
    src = edge_index[0].astype(jnp.int32)
    dst = edge_index[1].astype(jnp.int32)
    w = edge_weight.astype(jnp.float32)
    dst3 = dst.reshape(NW, NCHUNK_D, CH)
    w3 = w.reshape(NW, NCHUNK_D, CH)
    pk = jnp.stack(
        [src.reshape(NS, NCHUNK, CH), dst.reshape(NS, NCHUNK, CH)],
        axis=2)
    wp = w.reshape(NS, NCHUNK, CH)

    degp = _deg_call(dst3, w3)
    degp3 = degp.reshape(NC, NPAD, 1)
    xs2, dinvp = _tc1(Y, W1, degp3)
    dinv1 = dinvp.reshape(NPAD)
    acc2, _hs = _fused_call(xs2, dinv1, b1, pk, wp)
    mu, lv = _tc3(acc2, dinvp, W_mu, b_mu, W_lv, b_lv)
    return (mu, lv)


# speedup vs baseline: 1.1059x; 1.0992x over previous
import functools

import jax
import jax.numpy as jnp
from jax import lax
from jax.experimental import pallas as pl
from jax.experimental.pallas import tpu as pltpu
from jax.experimental.pallas import tpu_sc as plsc

N = 10000
E = 320000
D = 128
HD = D // 2
NC, NS = 2, 16
CH = 80
NB = 5
ECS = E // NS
NCHUNK = ECS // CH
NPAD = 10240
RPT = NPAD // NS
HBLK = 128
DRT = NPAD // NC // NS

_mesh = plsc.VectorSubcoreMesh(core_axis_name="c", subcore_axis_name="s")
_sc_params = pltpu.CompilerParams(use_tc_tiling_on_sc=False)


def _deg_body(dst3_hbm, wp_hbm, out_hbm, dstall, wall, zv, dl, rep, dsem,
              degsp):
    c = lax.axis_index("c")
    s = lax.axis_index("s")
    for i in range(RPT // 16):
        zv[pl.ds(i * 16, 16)] = jnp.zeros((16,), jnp.float32)
    pltpu.sync_copy(zv, degsp.at[pl.ds(s * RPT, RPT)])
    pltpu.sync_copy(dst3_hbm.at[s], dstall)
    pltpu.sync_copy(wp_hbm.at[s], wall)
    plsc.subcore_barrier()

    def fire(g, _):
        pltpu.async_copy(wall.at[g], degsp.at[dstall.at[g]], dsem, add=True)

        @pl.when(g >= 16)
        def _wait():
            pltpu.make_async_copy(wall.at[0], degsp.at[pl.ds(0, CH)],
                                  dsem).wait()
        return 0

    lax.fori_loop(0, NCHUNK, fire, 0)
    for _ in range(16):
        pltpu.make_async_copy(wall.at[0], degsp.at[pl.ds(0, CH)], dsem).wait()
    plsc.subcore_barrier()
    base = c * (NPAD // NC) + s * DRT
    pltpu.sync_copy(degsp.at[pl.ds(base, DRT)], dl)

    for blk in range(DRT // 80):
        def repl(t, _):
            dref = dl[pl.ds(blk * 80 + t * 16, 16)]
            for l in range(16):
                dj = dref[l]
                j = t * 16 + l
                for k in range(8):
                    rep[j, pl.ds(k * 16, 16)] = jnp.full((16,), dj,
                                                         jnp.float32)
            return 0

        lax.fori_loop(0, 5, repl, 0)
        pltpu.sync_copy(rep, out_hbm.at[pl.ds(base + blk * 80, 80)])


_deg_call = functools.partial(
    pl.kernel,
    out_type=jax.ShapeDtypeStruct((NPAD, D), jnp.float32),
    mesh=_mesh,
    scratch_types=[
        pltpu.VMEM((NCHUNK, CH), jnp.int32),
        pltpu.VMEM((NCHUNK, CH), jnp.float32),
        pltpu.VMEM((RPT,), jnp.float32),
        pltpu.VMEM((DRT,), jnp.float32),
        pltpu.VMEM((80, D), jnp.float32),
        pltpu.SemaphoreType.DMA,
        pltpu.VMEM_SHARED((NPAD,), jnp.float32),
    ],
    compiler_params=_sc_params,
)(_deg_body)


def _fused_body(xs_hbm, dinv_hbm, b1_hbm, src3_hbm, dst3_hbm, wp_hbm,
                out_hbm, tb_hbm, *rest):
    sv = rest[0:NB]
    dv = rest[NB:2 * NB]
    wv = rest[2 * NB:3 * NB]
    rows = rest[3 * NB:4 * NB]
    isem = rest[4 * NB:5 * NB]
    gsem = rest[5 * NB:6 * NB]
    ssem = rest[6 * NB:7 * NB]
    b1v, hb, dlb, accsp = rest[7 * NB:7 * NB + 4]
    c = lax.axis_index("c")
    s = lax.axis_index("s")
    r0 = s * RPT

    def start_idx(g, b):
        pltpu.async_copy(src3_hbm.at[s, g], sv[b], isem[b])
        pltpu.async_copy(dst3_hbm.at[s, g], dv[b], isem[b])
        pltpu.async_copy(wp_hbm.at[s, g], wv[b], isem[b])

    def wait_idx(b):
        for _ in range(3):
            pltpu.make_async_copy(src3_hbm.at[s, 0], sv[b], isem[b]).wait()

    def wait_rows_sem(sem, b):
        pltpu.make_async_copy(tb_hbm.at[0, pl.ds(0, CH)], rows[b],
                              sem[b]).wait()

    def scale(b):
        def sixteen(t, _):
            wrow = wv[b][pl.ds(t * 16, 16)]
            for l in range(16):
                wj = wrow[l]
                j = t * 16 + l
                for k in range(HD // 16):
                    sl = pl.ds(k * 16, 16)
                    rows[b][j, sl] = rows[b][j, sl] * wj
            return 0

        lax.fori_loop(0, CH // 16, sixteen, 0)

    def run_pass():
        start_idx(0, 0)
        start_idx(1, 1)
        wait_idx(0)
        pltpu.async_copy(tb_hbm.at[c].at[sv[0]], rows[0], gsem[0])

        def group(grp, _):
            for b in range(NB):
                g = grp * NB + b
                b2 = (b + 1) % NB
                b3 = (b + 2) % NB

                @pl.when(g + 2 < NCHUNK)
                def _prefetch_idx():
                    @pl.when(g >= NB - 2)
                    def _free():
                        wait_rows_sem(ssem, b3)
                    start_idx(g + 2, b3)

                @pl.when(g + 1 < NCHUNK)
                def _prefetch_rows():
                    wait_idx(b2)
                    pltpu.async_copy(tb_hbm.at[c].at[sv[b2]], rows[b2],
                                     gsem[b2])

                wait_rows_sem(gsem, b)
                scale(b)
                pltpu.async_copy(rows[b], accsp.at[dv[b]], ssem[b],
                                 add=True)
            return 0

        lax.fori_loop(0, NCHUNK // NB, group, 0)
        for b in range(NB):
            wait_rows_sem(ssem, b)

    pltpu.sync_copy(b1_hbm.at[pl.ds(c * HD, HD)], b1v)
    for blk in range(RPT // HBLK):
        rowbase = r0 + blk * HBLK
        pltpu.sync_copy(
            xs_hbm.at[pl.ds(rowbase, HBLK), pl.ds(c * HD, HD)], hb)
        pltpu.sync_copy(hb, tb_hbm.at[c, pl.ds(rowbase, HBLK)])
        pltpu.sync_copy(hb, accsp.at[pl.ds(rowbase, HBLK)])
    plsc.subcore_barrier()

    run_pass()
    plsc.subcore_barrier()

    for blk in range(RPT // HBLK):
        rowbase = r0 + blk * HBLK
        pltpu.sync_copy(accsp.at[pl.ds(rowbase, HBLK)], hb)
        pltpu.sync_copy(
            dinv_hbm.at[pl.ds(rowbase, HBLK), pl.ds(0, 16)], dlb)

        def rowloop(t, _):
            for l in range(16):
                j = t * 16 + l
                djv = dlb[j, pl.ds(0, 16)]
                for k in range(HD // 16):
                    sl = pl.ds(k * 16, 16)
                    v = hb[j, sl] * djv + b1v[sl]
                    hb[j, sl] = jnp.maximum(v, 0.0) * djv
            return 0

        lax.fori_loop(0, HBLK // 16, rowloop, 0)
        pltpu.sync_copy(hb, tb_hbm.at[c, pl.ds(rowbase, HBLK)])
        pltpu.sync_copy(hb, accsp.at[pl.ds(rowbase, HBLK)])
    plsc.subcore_barrier()

    run_pass()
    plsc.subcore_barrier()
    pltpu.sync_copy(accsp.at[pl.ds(r0, RPT)],
                    out_hbm.at[pl.ds(r0, RPT), pl.ds(c * HD, HD)])


_fused_call = functools.partial(
    pl.kernel,
    out_type=(
        jax.ShapeDtypeStruct((NPAD, D), jnp.float32),
        jax.ShapeDtypeStruct((NC, NPAD, HD), jnp.float32),
    ),
    mesh=_mesh,
    scratch_types=(
        [pltpu.VMEM((CH,), jnp.int32) for _ in range(NB)]
        + [pltpu.VMEM((CH,), jnp.int32) for _ in range(NB)]
        + [pltpu.VMEM((CH,), jnp.float32) for _ in range(NB)]
        + [pltpu.VMEM((CH, HD), jnp.float32) for _ in range(NB)]
        + [pltpu.SemaphoreType.DMA for _ in range(3 * NB)]
        + [
            pltpu.VMEM((HD,), jnp.float32),
            pltpu.VMEM((HBLK, HD), jnp.float32),
            pltpu.VMEM((HBLK, 16), jnp.float32),
            pltpu.VMEM_SHARED((NPAD, HD), jnp.float32),
        ]
    ),
    compiler_params=_sc_params,
)(_fused_body)


def _tc1_body(y_ref, w1_ref, degrep_ref, xs_ref, dinv_ref):
    xw = jnp.dot(y_ref[...], w1_ref[...], preferred_element_type=jnp.float32)
    deg = degrep_ref[...] + 1.0
    dinv = jnp.where(deg > 0, lax.rsqrt(deg), 0.0)
    dinv_ref[...] = dinv
    xs_ref[:N, :] = xw * dinv[:N, :]
    xs_ref[pl.ds(N, NPAD - N), :] = jnp.zeros((NPAD - N, D), jnp.float32)


def _tc1(Y, W1, degrep):
    return pl.pallas_call(
        _tc1_body,
        out_shape=[
            jax.ShapeDtypeStruct((NPAD, D), jnp.float32),
            jax.ShapeDtypeStruct((NPAD, D), jnp.float32),
        ],
    )(Y, W1, degrep)


def _tc3_body(acc_ref, dinv_ref, wmu_ref, bmu_ref, wlv_ref, blv_ref,
              mu_ref, lv_ref):
    p2 = acc_ref[:N, :] * dinv_ref[:N, :]
    mu_ref[...] = jnp.dot(p2, wmu_ref[...],
                          preferred_element_type=jnp.float32) + bmu_ref[...]
    lv_ref[...] = jnp.dot(p2, wlv_ref[...],
                          preferred_element_type=jnp.float32) + blv_ref[...]


def _tc3(acc, dinvrep, W_mu, b_mu, W_lv, b_lv):
    lat = W_mu.shape[1]
    return pl.pallas_call(
        _tc3_body,
        out_shape=[
            jax.ShapeDtypeStruct((N, lat), jnp.float32),
            jax.ShapeDtypeStruct((N, lat), jnp.float32),
        ],
    )(acc, dinvrep, W_mu, b_mu, W_lv, b_lv)


@jax.jit
def kernel(Y, edge_index, edge_weight, W1, b1, W_mu, b_mu, W_lv, b_lv):
    src3 = edge_index[0].astype(jnp.int32).reshape(NS, NCHUNK, CH)
    dst3 = edge_index[1].astype(jnp.int32).reshape(NS, NCHUNK, CH)
    wp = edge_weight.astype(jnp.float32).reshape(NS, NCHUNK, CH)

    degrep = _deg_call(dst3, wp)
    xs, dinvrep = _tc1(Y, W1, degrep)
    s2, _tb = _fused_call(xs, dinvrep, b1, src3, dst3, wp)
    mu, lv = _tc3(s2, dinvrep, W_mu, b_mu, W_lv, b_lv)
    return (mu, lv)


# speedup vs baseline: 1.1411x; 1.0318x over previous
import functools

import jax
import jax.numpy as jnp
from jax import lax
from jax.experimental import pallas as pl
from jax.experimental.pallas import tpu as pltpu
from jax.experimental.pallas import tpu_sc as plsc

N = 10000
E = 320000
D = 128
HD = D // 2
NC, NS = 2, 16
CH = 80
NB = 5
ECS = E // NS
NCHUNK = ECS // CH
NPAD = 10240
RPT = NPAD // NS
HBLK = 128
DRT = NPAD // NC // NS

_mesh = plsc.VectorSubcoreMesh(core_axis_name="c", subcore_axis_name="s")
_sc_params = pltpu.CompilerParams(use_tc_tiling_on_sc=False)


def _deg_body(ei_hbm, wp_hbm, out_hbm, dstall, wall, zv, dl, rep, dsem,
              degsp):
    c = lax.axis_index("c")
    s = lax.axis_index("s")
    for i in range(RPT // 16):
        zv[pl.ds(i * 16, 16)] = jnp.zeros((16,), jnp.float32)
    pltpu.sync_copy(zv, degsp.at[pl.ds(s * RPT, RPT)])
    pltpu.sync_copy(ei_hbm.at[1, pl.ds(s * ECS, ECS)], dstall)
    pltpu.sync_copy(wp_hbm.at[pl.ds(s * ECS, ECS)], wall)
    plsc.subcore_barrier()

    def fire(g, _):
        pltpu.async_copy(wall.at[pl.ds(g * CH, CH)],
                         degsp.at[dstall.at[pl.ds(g * CH, CH)]],
                         dsem, add=True)

        @pl.when(g >= 16)
        def _wait():
            pltpu.make_async_copy(wall.at[pl.ds(0, CH)],
                                  degsp.at[pl.ds(0, CH)], dsem).wait()
        return 0

    lax.fori_loop(0, NCHUNK, fire, 0)
    for _ in range(16):
        pltpu.make_async_copy(wall.at[pl.ds(0, CH)], degsp.at[pl.ds(0, CH)],
                              dsem).wait()
    plsc.subcore_barrier()
    base = c * (NPAD // NC) + s * DRT
    pltpu.sync_copy(degsp.at[pl.ds(base, DRT)], dl)

    for blk in range(DRT // 80):
        def repl(t, _):
            dref = dl[pl.ds(blk * 80 + t * 16, 16)]
            for l in range(16):
                dj = dref[l]
                j = t * 16 + l
                for k in range(8):
                    rep[j, pl.ds(k * 16, 16)] = jnp.full((16,), dj,
                                                         jnp.float32)
            return 0

        lax.fori_loop(0, 5, repl, 0)
        pltpu.sync_copy(rep, out_hbm.at[pl.ds(base + blk * 80, 80)])


_deg_call = functools.partial(
    pl.kernel,
    out_type=jax.ShapeDtypeStruct((NPAD, D), jnp.float32),
    mesh=_mesh,
    scratch_types=[
        pltpu.VMEM((ECS,), jnp.int32),
        pltpu.VMEM((ECS,), jnp.float32),
        pltpu.VMEM((RPT,), jnp.float32),
        pltpu.VMEM((DRT,), jnp.float32),
        pltpu.VMEM((80, D), jnp.float32),
        pltpu.SemaphoreType.DMA,
        pltpu.VMEM_SHARED((NPAD,), jnp.float32),
    ],
    compiler_params=_sc_params,
)(_deg_body)


def _fused_body(xs_hbm, dinv_hbm, b1_hbm, ei_hbm, wp_hbm,
                out_hbm, tb_hbm, *rest):
    eiv = rest[0:NB]
    wv = rest[NB:2 * NB]
    rows = rest[2 * NB:3 * NB]
    isem = rest[3 * NB:4 * NB]
    gsem = rest[4 * NB:5 * NB]
    ssem = rest[5 * NB:6 * NB]
    b1v, hb, dlb, accsp = rest[6 * NB:6 * NB + 4]
    c = lax.axis_index("c")
    s = lax.axis_index("s")
    r0 = s * RPT

    def start_idx(g, b):
        off = s * ECS + g * CH
        pltpu.async_copy(ei_hbm.at[:, pl.ds(off, CH)], eiv[b], isem[b])
        pltpu.async_copy(wp_hbm.at[pl.ds(off, CH)], wv[b], isem[b])

    def wait_idx(b):
        pltpu.make_async_copy(ei_hbm.at[:, pl.ds(0, CH)], eiv[b],
                              isem[b]).wait()
        pltpu.make_async_copy(wp_hbm.at[pl.ds(0, CH)], wv[b],
                              isem[b]).wait()

    def wait_rows_sem(sem, b):
        pltpu.make_async_copy(tb_hbm.at[0, pl.ds(0, CH)], rows[b],
                              sem[b]).wait()

    def scale(b):
        def sixteen(t, _):
            wrow = wv[b][pl.ds(t * 16, 16)]
            for l in range(16):
                wj = wrow[l]
                j = t * 16 + l
                for k in range(HD // 16):
                    sl = pl.ds(k * 16, 16)
                    rows[b][j, sl] = rows[b][j, sl] * wj
            return 0

        lax.fori_loop(0, CH // 16, sixteen, 0)

    def run_pass():
        start_idx(0, 0)
        start_idx(1, 1)
        wait_idx(0)
        pltpu.async_copy(tb_hbm.at[c].at[eiv[0].at[0]], rows[0], gsem[0])

        def group(grp, _):
            for b in range(NB):
                g = grp * NB + b
                b2 = (b + 1) % NB
                b3 = (b + 2) % NB

                @pl.when(g + 2 < NCHUNK)
                def _prefetch_idx():
                    @pl.when(g >= NB - 2)
                    def _free():
                        wait_rows_sem(ssem, b3)
                    start_idx(g + 2, b3)

                @pl.when(g + 1 < NCHUNK)
                def _prefetch_rows():
                    wait_idx(b2)
                    pltpu.async_copy(tb_hbm.at[c].at[eiv[b2].at[0]],
                                     rows[b2], gsem[b2])

                wait_rows_sem(gsem, b)
                scale(b)
                pltpu.async_copy(rows[b], accsp.at[eiv[b].at[1]], ssem[b],
                                 add=True)
            return 0

        lax.fori_loop(0, NCHUNK // NB, group, 0)
        for b in range(NB):
            wait_rows_sem(ssem, b)

    pltpu.sync_copy(b1_hbm.at[pl.ds(c * HD, HD)], b1v)
    for blk in range(RPT // HBLK):
        rowbase = r0 + blk * HBLK
        pltpu.sync_copy(
            xs_hbm.at[pl.ds(rowbase, HBLK), pl.ds(c * HD, HD)], hb)
        pltpu.sync_copy(hb, tb_hbm.at[c, pl.ds(rowbase, HBLK)])
        pltpu.sync_copy(hb, accsp.at[pl.ds(rowbase, HBLK)])
    plsc.subcore_barrier()

    run_pass()
    plsc.subcore_barrier()

    for blk in range(RPT // HBLK):
        rowbase = r0 + blk * HBLK
        pltpu.sync_copy(accsp.at[pl.ds(rowbase, HBLK)], hb)
        pltpu.sync_copy(
            dinv_hbm.at[pl.ds(rowbase, HBLK), pl.ds(0, 16)], dlb)

        def rowloop(t, _):
            for l in range(16):
                j = t * 16 + l
                djv = dlb[j, pl.ds(0, 16)]
                for k in range(HD // 16):
                    sl = pl.ds(k * 16, 16)
                    v = hb[j, sl] * djv + b1v[sl]
                    hb[j, sl] = jnp.maximum(v, 0.0) * djv
            return 0

        lax.fori_loop(0, HBLK // 16, rowloop, 0)
        pltpu.sync_copy(hb, tb_hbm.at[c, pl.ds(rowbase, HBLK)])
        pltpu.sync_copy(hb, accsp.at[pl.ds(rowbase, HBLK)])
    plsc.subcore_barrier()

    run_pass()
    plsc.subcore_barrier()
    pltpu.sync_copy(accsp.at[pl.ds(r0, RPT)],
                    out_hbm.at[pl.ds(r0, RPT), pl.ds(c * HD, HD)])


_fused_call = functools.partial(
    pl.kernel,
    out_type=(
        jax.ShapeDtypeStruct((NPAD, D), jnp.float32),
        jax.ShapeDtypeStruct((NC, NPAD, HD), jnp.float32),
    ),
    mesh=_mesh,
    scratch_types=(
        [pltpu.VMEM((2, CH), jnp.int32) for _ in range(NB)]
        + [pltpu.VMEM((CH,), jnp.float32) for _ in range(NB)]
        + [pltpu.VMEM((CH, HD), jnp.float32) for _ in range(NB)]
        + [pltpu.SemaphoreType.DMA for _ in range(3 * NB)]
        + [
            pltpu.VMEM((HD,), jnp.float32),
            pltpu.VMEM((HBLK, HD), jnp.float32),
            pltpu.VMEM((HBLK, 16), jnp.float32),
            pltpu.VMEM_SHARED((NPAD, HD), jnp.float32),
        ]
    ),
    compiler_params=_sc_params,
)(_fused_body)


def _tc1_body(y_ref, w1_ref, degrep_ref, xs_ref, dinv_ref):
    xw = jnp.dot(y_ref[...], w1_ref[...], preferred_element_type=jnp.float32)
    deg = degrep_ref[...] + 1.0
    dinv = jnp.where(deg > 0, lax.rsqrt(deg), 0.0)
    dinv_ref[...] = dinv
    xs_ref[:N, :] = xw * dinv[:N, :]
    xs_ref[pl.ds(N, NPAD - N), :] = jnp.zeros((NPAD - N, D), jnp.float32)


def _tc1(Y, W1, degrep):
    return pl.pallas_call(
        _tc1_body,
        out_shape=[
            jax.ShapeDtypeStruct((NPAD, D), jnp.float32),
            jax.ShapeDtypeStruct((NPAD, D), jnp.float32),
        ],
    )(Y, W1, degrep)


def _tc3_body(acc_ref, dinv_ref, wmu_ref, bmu_ref, wlv_ref, blv_ref,
              mu_ref, lv_ref):
    p2 = acc_ref[:N, :] * dinv_ref[:N, :]
    mu_ref[...] = jnp.dot(p2, wmu_ref[...],
                          preferred_element_type=jnp.float32) + bmu_ref[...]
    lv_ref[...] = jnp.dot(p2, wlv_ref[...],
                          preferred_element_type=jnp.float32) + blv_ref[...]


def _tc3(acc, dinvrep, W_mu, b_mu, W_lv, b_lv):
    lat = W_mu.shape[1]
    return pl.pallas_call(
        _tc3_body,
        out_shape=[
            jax.ShapeDtypeStruct((N, lat), jnp.float32),
            jax.ShapeDtypeStruct((N, lat), jnp.float32),
        ],
    )(acc, dinvrep, W_mu, b_mu, W_lv, b_lv)


@jax.jit
def kernel(Y, edge_index, edge_weight, W1, b1, W_mu, b_mu, W_lv, b_lv):
    ei = edge_index.astype(jnp.int32)
    wp = edge_weight.astype(jnp.float32)

    degrep = _deg_call(ei, wp)
    xs, dinvrep = _tc1(Y, W1, degrep)
    s2, _tb = _fused_call(xs, dinvrep, b1, ei, wp)
    mu, lv = _tc3(s2, dinvrep, W_mu, b_mu, W_lv, b_lv)
    return (mu, lv)
